# baseline (speedup 1.0000x reference)
import functools

import jax
import jax.numpy as jnp
from jax.experimental import pallas as pl
from jax.experimental.pallas import tpu as pltpu
from jax.experimental.pallas import tpu_sc as plsc

_G = 1024
_K = 32


def _fps_body(xyzT_ref, idx_ref):
    X = xyzT_ref[0]
    Y = xyzT_ref[1]
    Z = xyzT_ref[2]
    B, N = X.shape
    lane = jax.lax.broadcasted_iota(jnp.int32, (B, N), 1)
    lane_h = jax.lax.broadcasted_iota(jnp.int32, (B, _G), 1)

    sub_h = jax.lax.broadcasted_iota(jnp.int32, (B, _G), 0)
    zero_h = lane_h * 0 + sub_h * 0

    def body(i, carry):
        dist_min, far = carry
        far_b = far + zero_h
        idx_ref[...] = jnp.where(lane_h == i, far_b, idx_ref[...])
        eq = lane == far
        cx = jnp.sum(jnp.where(eq, X, 0.0), axis=1, keepdims=True)
        cy = jnp.sum(jnp.where(eq, Y, 0.0), axis=1, keepdims=True)
        cz = jnp.sum(jnp.where(eq, Z, 0.0), axis=1, keepdims=True)
        d = (X - cx) ** 2 + (Y - cy) ** 2 + (Z - cz) ** 2
        dist_min = jnp.minimum(dist_min, d)
        m = jnp.max(dist_min, axis=1, keepdims=True)
        far_new = jnp.min(jnp.where(dist_min == m, lane, N),
                          axis=1, keepdims=True)
        return dist_min, far_new

    dist0 = jnp.full((B, N), 1e10, jnp.float32)
    far0 = jnp.zeros((B, 1), jnp.int32)
    jax.lax.fori_loop(0, _G, body, (dist0, far0), unroll=8)


def _fps_pallas(xyz):
    B, N, _ = xyz.shape
    xyzT = jnp.transpose(xyz, (2, 0, 1))
    return pl.pallas_call(
        _fps_body,
        out_shape=jax.ShapeDtypeStruct((B, _G), jnp.int32),
    )(xyzT)


_KT = 512


def _knn_body(lc_ref, xyzB_ref, out_ref):
    X = xyzB_ref[0, 0:1, :]
    Y = xyzB_ref[0, 1:2, :]
    Z = xyzB_ref[0, 2:3, :]
    Xc = lc_ref[:, 64:65]
    Yc = lc_ref[:, 65:66]
    Zc = lc_ref[:, 66:67]
    N = X.shape[1]
    csq = (Xc * Xc + Yc * Yc) + Zc * Zc
    psq = (X * X + Y * Y) + Z * Z

    def _bf(v):
        return v.astype(jnp.bfloat16).astype(jnp.float32)

    dot = (_bf(X) * _bf(Xc) + _bf(Y) * _bf(Yc)) + _bf(Z) * _bf(Zc)
    D = (csq + psq) - 2.0 * dot

    lane = jax.lax.broadcasted_iota(jnp.int32, (_KT, N), 1)
    lane_o = jax.lax.broadcasted_iota(jnp.int32, (_KT, _K), 1)
    sub_o = jax.lax.broadcasted_iota(jnp.int32, (_KT, _K), 0)
    zero_o = lane_o * 0 + sub_o * 0
    inf = jnp.float32(jnp.inf)

    def ext(k, D):
        m = jnp.min(D, axis=1, keepdims=True)
        idx = jnp.min(jnp.where(D == m, lane, N), axis=1, keepdims=True)
        out_ref[...] = jnp.where(lane_o == k, idx + zero_o, out_ref[...])
        return jnp.where(lane == idx, inf, D)

    jax.lax.fori_loop(0, _K, ext, D, unroll=16)


def _knn_pallas(lcg, xyzB):
    BG = lcg.shape[0]
    B, _, N = xyzB.shape
    tiles_per_b = _G // _KT
    return pl.pallas_call(
        _knn_body,
        grid=(BG // _KT,),
        in_specs=[
            pl.BlockSpec((_KT, 128), lambda t: (t, 0)),
            pl.BlockSpec((1, 3, N), lambda t: (t // tiles_per_b, 0, 0)),
        ],
        out_specs=pl.BlockSpec((_KT, _K), lambda t: (t, 0)),
        out_shape=jax.ShapeDtypeStruct((BG, _K), jnp.int32),
        compiler_params=pltpu.CompilerParams(
            dimension_semantics=("parallel",)),
    )(lcg, xyzB)


_W = 128


def _sc_gather(table, idx):
    V, D = table.shape
    M = idx.shape[0]
    idx2 = idx.reshape(1, M)
    mesh = plsc.VectorSubcoreMesh(core_axis_name="c", subcore_axis_name="s")

    @functools.partial(
        pl.kernel, mesh=mesh,
        out_type=jax.ShapeDtypeStruct((M, D), table.dtype))
    def k(table_hbm, idx_hbm, out_hbm):
        def body(i_vmem, o_vmem):
            pltpu.sync_copy(table_hbm.at[i_vmem.at[0]], o_vmem)

        pltpu.emit_pipeline(
            body,
            grid=(M // _W,),
            in_specs=[pl.BlockSpec((1, _W), index_map=lambda i: (0, i))],
            out_specs=[pl.BlockSpec((_W, D), index_map=lambda i: (i, 0))],
            core_axis_name=("c", "s"),
            dimension_semantics=(pltpu.PARALLEL,),
        )(idx_hbm, out_hbm)

    return k(table, idx2)


_TN = 256


def _sum_body(g_ref, lc_ref, out_ref):
    knn3 = g_ref[:, :, 64:67]
    lc3 = lc_ref[:, 64:67][:, None, :]
    diff = knn3 - lc3
    s1 = jnp.sum(diff)
    s2 = jnp.sum(diff * diff)
    lane = jax.lax.broadcasted_iota(jnp.int32, (1, 128), 1)
    part = jnp.where(lane == 0, s1, jnp.where(lane == 1, s2, 0.0))

    @pl.when(pl.program_id(0) == 0)
    def _():
        out_ref[...] = jnp.zeros_like(out_ref)

    out_ref[...] += part


def _finish_body(g_ref, lc_ref, scale_ref, nrm_ref, out_ref):
    g = g_ref[...]
    lc = lc_ref[...]
    scale = scale_ref[...][:, :, None]
    lane = jax.lax.broadcasted_iota(jnp.int32, g.shape, 2)
    lc3 = lc[:, None, :]
    coord = (lane >= 64) & (lane < 67)
    nrm_ref[...] = jnp.where(coord, (g - lc3) * scale, 0.0)
    lcsh = jnp.concatenate([lc[:, 64:], lc[:, :64]], axis=1)
    out_ref[...] = jnp.where(lane < 64, g, lcsh[:, None, :])


def _norm_pallas(g3, lcg):
    BG = g3.shape[0]
    n = jnp.float32(BG * _K * 3)
    sums = pl.pallas_call(
        _sum_body,
        grid=(BG // _TN,),
        in_specs=[
            pl.BlockSpec((_TN, _K, 128), lambda t: (t, 0, 0)),
            pl.BlockSpec((_TN, 128), lambda t: (t, 0)),
        ],
        out_specs=pl.BlockSpec((1, 128), lambda t: (0, 0)),
        out_shape=jax.ShapeDtypeStruct((1, 128), jnp.float32),
    )(g3, lcg)
    s1, s2 = sums[0, 0], sums[0, 1]
    mean = s1 / n
    var = (s2 - n * mean * mean) / (n - 1.0)
    scale = (1.0 / (jnp.sqrt(var) + 1e-05)).reshape(1, 1)
    return pl.pallas_call(
        _finish_body,
        grid=(BG // _TN,),
        in_specs=[
            pl.BlockSpec((_TN, _K, 128), lambda t: (t, 0, 0)),
            pl.BlockSpec((_TN, 128), lambda t: (t, 0)),
            pl.BlockSpec((1, 1), lambda t: (0, 0)),
        ],
        out_specs=[
            pl.BlockSpec((_TN, _K, 128), lambda t: (t, 0, 0)),
            pl.BlockSpec((_TN, _K, 128), lambda t: (t, 0, 0)),
        ],
        out_shape=[
            jax.ShapeDtypeStruct(g3.shape, jnp.float32),
            jax.ShapeDtypeStruct(g3.shape, jnp.float32),
        ],
        compiler_params=pltpu.CompilerParams(
            dimension_semantics=("parallel",)),
    )(g3, lcg, scale)


def kernel(xyz, x):
    B, N, _ = xyz.shape
    C = x.shape[-1]
    fps_idx = _fps_pallas(xyz)
    base = (jnp.arange(B, dtype=jnp.int32) * N)[:, None]
    fps_flat = (fps_idx + base).reshape(B * _G)
    xcat = jnp.concatenate(
        [x, xyz, jnp.zeros((B, N, 128 - C - 3), jnp.float32)],
        axis=-1).reshape(B * N, 128)
    lcg = _sc_gather(xcat, fps_flat)
    lc_xyz = lcg[:, C:C + 3].reshape(B, _G, 3)
    lc_x = lcg[:, :C].reshape(B, _G, C)
    xyzB = jnp.transpose(xyz, (0, 2, 1))
    knn_idx = _knn_pallas(lcg, xyzB)
    knn_flat = (knn_idx.reshape(B, _G * _K) + base).reshape(B * _G * _K)
    g3 = _sc_gather(xcat, knn_flat).reshape(B * _G, _K, 128)
    nrm, knn_x3 = _norm_pallas(g3, lcg)
    knn_xyz = nrm.reshape(B, _G, _K, 128)[..., C:C + 3]
    knn_x = knn_x3.reshape(B, _G, _K, 128)
    return (lc_xyz, lc_x, knn_xyz, knn_x)


# speedup vs baseline: 1.0412x; 1.0412x over previous
import functools

import jax
import jax.numpy as jnp
from jax.experimental import pallas as pl
from jax.experimental.pallas import tpu as pltpu
from jax.experimental.pallas import tpu_sc as plsc

_G = 1024
_K = 32


def _fps_body(xyzT_ref, idx_ref):
    X = xyzT_ref[0]
    Y = xyzT_ref[1]
    Z = xyzT_ref[2]
    B, N = X.shape
    lane = jax.lax.broadcasted_iota(jnp.int32, (B, N), 1)
    lane_h = jax.lax.broadcasted_iota(jnp.int32, (B, _G), 1)

    sub_h = jax.lax.broadcasted_iota(jnp.int32, (B, _G), 0)
    zero_h = lane_h * 0 + sub_h * 0

    def body(i, carry):
        dist_min, far = carry
        far_b = far + zero_h
        idx_ref[...] = jnp.where(lane_h == i, far_b, idx_ref[...])
        eq = lane == far
        cx = jnp.sum(jnp.where(eq, X, 0.0), axis=1, keepdims=True)
        cy = jnp.sum(jnp.where(eq, Y, 0.0), axis=1, keepdims=True)
        cz = jnp.sum(jnp.where(eq, Z, 0.0), axis=1, keepdims=True)
        d = (X - cx) ** 2 + (Y - cy) ** 2 + (Z - cz) ** 2
        dist_min = jnp.minimum(dist_min, d)
        m = jnp.max(dist_min, axis=1, keepdims=True)
        far_new = jnp.min(jnp.where(dist_min == m, lane, N),
                          axis=1, keepdims=True)
        return dist_min, far_new

    dist0 = jnp.full((B, N), 1e10, jnp.float32)
    far0 = jnp.zeros((B, 1), jnp.int32)
    jax.lax.fori_loop(0, _G, body, (dist0, far0), unroll=4)


def _fps_pallas(xyz):
    B, N, _ = xyz.shape
    xyzT = jnp.transpose(xyz, (2, 0, 1))
    return pl.pallas_call(
        _fps_body,
        out_shape=jax.ShapeDtypeStruct((B, _G), jnp.int32),
    )(xyzT)


_KT = 256


def _knn_body(lc_ref, xyzB_ref, out_ref):
    X = xyzB_ref[0, 0:1, :]
    Y = xyzB_ref[0, 1:2, :]
    Z = xyzB_ref[0, 2:3, :]
    Xc = lc_ref[:, 64:65]
    Yc = lc_ref[:, 65:66]
    Zc = lc_ref[:, 66:67]
    N = X.shape[1]
    csq = (Xc * Xc + Yc * Yc) + Zc * Zc
    psq = (X * X + Y * Y) + Z * Z

    def _bf(v):
        return v.astype(jnp.bfloat16).astype(jnp.float32)

    dot = (_bf(X) * _bf(Xc) + _bf(Y) * _bf(Yc)) + _bf(Z) * _bf(Zc)
    D = (csq + psq) - 2.0 * dot

    lane = jax.lax.broadcasted_iota(jnp.int32, (_KT, N), 1)
    lane_o = jax.lax.broadcasted_iota(jnp.int32, (_KT, _K), 1)
    sub_o = jax.lax.broadcasted_iota(jnp.int32, (_KT, _K), 0)
    zero_o = lane_o * 0 + sub_o * 0
    inf = jnp.float32(jnp.inf)

    def ext(k, D):
        m = jnp.min(D, axis=1, keepdims=True)
        idx = jnp.min(jnp.where(D == m, lane, N), axis=1, keepdims=True)
        out_ref[...] = jnp.where(lane_o == k, idx + zero_o, out_ref[...])
        return jnp.where(lane == idx, inf, D)

    jax.lax.fori_loop(0, _K, ext, D, unroll=16)


def _knn_pallas(lcg, xyzB):
    BG = lcg.shape[0]
    B, _, N = xyzB.shape
    tiles_per_b = _G // _KT
    return pl.pallas_call(
        _knn_body,
        grid=(BG // _KT,),
        in_specs=[
            pl.BlockSpec((_KT, 128), lambda t: (t, 0)),
            pl.BlockSpec((1, 3, N), lambda t: (t // tiles_per_b, 0, 0)),
        ],
        out_specs=pl.BlockSpec((_KT, _K), lambda t: (t, 0)),
        out_shape=jax.ShapeDtypeStruct((BG, _K), jnp.int32),
        compiler_params=pltpu.CompilerParams(
            dimension_semantics=("parallel",)),
    )(lcg, xyzB)


_W = 128


def _sc_gather(table, idx):
    V, D = table.shape
    M = idx.shape[0]
    idx2 = idx.reshape(1, M)
    mesh = plsc.VectorSubcoreMesh(core_axis_name="c", subcore_axis_name="s")

    @functools.partial(
        pl.kernel, mesh=mesh,
        out_type=jax.ShapeDtypeStruct((M, D), table.dtype))
    def k(table_hbm, idx_hbm, out_hbm):
        def body(i_vmem, o_vmem):
            pltpu.sync_copy(table_hbm.at[i_vmem.at[0]], o_vmem)

        pltpu.emit_pipeline(
            body,
            grid=(M // _W,),
            in_specs=[pl.BlockSpec((1, _W), index_map=lambda i: (0, i))],
            out_specs=[pl.BlockSpec((_W, D), index_map=lambda i: (i, 0))],
            core_axis_name=("c", "s"),
            dimension_semantics=(pltpu.PARALLEL,),
        )(idx_hbm, out_hbm)

    return k(table, idx2)


_TN = 256


def _sum_body(g_ref, lc_ref, out_ref):
    knn3 = g_ref[:, :, 64:67]
    lc3 = lc_ref[:, 64:67][:, None, :]
    diff = knn3 - lc3
    s1 = jnp.sum(diff)
    s2 = jnp.sum(diff * diff)
    lane = jax.lax.broadcasted_iota(jnp.int32, (1, 128), 1)
    part = jnp.where(lane == 0, s1, jnp.where(lane == 1, s2, 0.0))

    @pl.when(pl.program_id(0) == 0)
    def _():
        out_ref[...] = jnp.zeros_like(out_ref)

    out_ref[...] += part


def _finish_body(g_ref, lc_ref, scale_ref, nrm_ref, out_ref):
    g = g_ref[...]
    lc = lc_ref[...]
    scale = scale_ref[...][:, :, None]
    lane = jax.lax.broadcasted_iota(jnp.int32, g.shape, 2)
    lc3 = lc[:, None, :]
    coord = (lane >= 64) & (lane < 67)
    nrm_ref[...] = jnp.where(coord, (g - lc3) * scale, 0.0)
    lcsh = jnp.concatenate([lc[:, 64:], lc[:, :64]], axis=1)
    out_ref[...] = jnp.where(lane < 64, g, lcsh[:, None, :])


def _norm_pallas(g3, lcg):
    BG = g3.shape[0]
    n = jnp.float32(BG * _K * 3)
    sums = pl.pallas_call(
        _sum_body,
        grid=(BG // _TN,),
        in_specs=[
            pl.BlockSpec((_TN, _K, 128), lambda t: (t, 0, 0)),
            pl.BlockSpec((_TN, 128), lambda t: (t, 0)),
        ],
        out_specs=pl.BlockSpec((1, 128), lambda t: (0, 0)),
        out_shape=jax.ShapeDtypeStruct((1, 128), jnp.float32),
    )(g3, lcg)
    s1, s2 = sums[0, 0], sums[0, 1]
    mean = s1 / n
    var = (s2 - n * mean * mean) / (n - 1.0)
    scale = (1.0 / (jnp.sqrt(var) + 1e-05)).reshape(1, 1)
    return pl.pallas_call(
        _finish_body,
        grid=(BG // _TN,),
        in_specs=[
            pl.BlockSpec((_TN, _K, 128), lambda t: (t, 0, 0)),
            pl.BlockSpec((_TN, 128), lambda t: (t, 0)),
            pl.BlockSpec((1, 1), lambda t: (0, 0)),
        ],
        out_specs=[
            pl.BlockSpec((_TN, _K, 128), lambda t: (t, 0, 0)),
            pl.BlockSpec((_TN, _K, 128), lambda t: (t, 0, 0)),
        ],
        out_shape=[
            jax.ShapeDtypeStruct(g3.shape, jnp.float32),
            jax.ShapeDtypeStruct(g3.shape, jnp.float32),
        ],
        compiler_params=pltpu.CompilerParams(
            dimension_semantics=("parallel",)),
    )(g3, lcg, scale)


def kernel(xyz, x):
    B, N, _ = xyz.shape
    C = x.shape[-1]
    fps_idx = _fps_pallas(xyz)
    base = (jnp.arange(B, dtype=jnp.int32) * N)[:, None]
    fps_flat = (fps_idx + base).reshape(B * _G)
    xcat = jnp.concatenate(
        [x, xyz, jnp.zeros((B, N, 128 - C - 3), jnp.float32)],
        axis=-1).reshape(B * N, 128)
    lcg = _sc_gather(xcat, fps_flat)
    lc_xyz = lcg[:, C:C + 3].reshape(B, _G, 3)
    lc_x = lcg[:, :C].reshape(B, _G, C)
    xyzB = jnp.transpose(xyz, (0, 2, 1))
    knn_idx = _knn_pallas(lcg, xyzB)
    knn_flat = (knn_idx.reshape(B, _G * _K) + base).reshape(B * _G * _K)
    g3 = _sc_gather(xcat, knn_flat).reshape(B * _G, _K, 128)
    nrm, knn_x3 = _norm_pallas(g3, lcg)
    knn_xyz = nrm.reshape(B, _G, _K, 128)[..., C:C + 3]
    knn_x = knn_x3.reshape(B, _G, _K, 128)
    return (lc_xyz, lc_x, knn_xyz, knn_x)
